# baseline (speedup 1.0000x reference)
import functools

import jax
import jax.numpy as jnp
from jax import lax
from jax.experimental import pallas as pl
from jax.experimental.pallas import tpu as pltpu
from jax.experimental.pallas import tpu_sc as plsc

TOKEN_UNIVERSE = 120000
NUM_OOV = 1
NUM_WORKERS = 32
LANES = 16
CHUNK = 1024
VCHUNK = 2000


def _unrolled(n_total, unroll, body):
    assert n_total % unroll == 0

    def outer(o, _):
        for u in range(unroll):
            body(o * jnp.int32(unroll) + jnp.int32(u))
        return _

    lax.fori_loop(jnp.int32(0), jnp.int32(n_total // unroll), outer, None)


def _sc_lookup(tokp, vocp, offv):
    n2 = tokp.shape[0]
    v = vocp.shape[0] // 2
    n = n2 // 2
    per_w = n // NUM_WORKERS
    n_chunks = per_w // CHUNK
    v_chunks = v // VCHUNK
    assert per_w * NUM_WORKERS == n and n_chunks * CHUNK == per_w
    assert v_chunks * VCHUNK == v

    mesh = plsc.VectorSubcoreMesh(
        core_axis_name="c", subcore_axis_name="s", num_cores=2, num_subcores=16
    )

    @functools.partial(
        pl.kernel,
        out_type=jax.ShapeDtypeStruct((n2,), jnp.int32),
        mesh=mesh,
        compiler_params=pltpu.CompilerParams(needs_layout_passes=False),
        scratch_types=[
            pltpu.VMEM((TOKEN_UNIVERSE,), jnp.int32),
            pltpu.VMEM((2 * VCHUNK,), jnp.int32),
            pltpu.VMEM((2 * CHUNK,), jnp.int32),
            pltpu.VMEM((2 * CHUNK,), jnp.int32),
            pltpu.VMEM((LANES,), jnp.int32),
        ],
    )
    def k(tok_hbm, voc_hbm, off_hbm, out_hbm, table_v, vbuf_v, inb_v, outb_v,
          offv_v):
        lane = lax.iota(jnp.int32, LANES)
        lane2 = lane * 2

        pltpu.sync_copy(off_hbm, offv_v)
        off = offv_v[...]

        zeros = jnp.zeros((LANES,), jnp.int32)

        def zero_body(i):
            table_v[pl.ds(i * LANES, LANES)] = zeros

        _unrolled(TOKEN_UNIVERSE // LANES, 10, zero_body)

        def zero_out_body(i):
            outb_v[pl.ds(i * LANES, LANES)] = zeros

        _unrolled(2 * CHUNK // LANES, 8, zero_out_body)

        def build_chunk(c, _):
            pltpu.sync_copy(voc_hbm.at[pl.ds(c * (2 * VCHUNK), 2 * VCHUNK)],
                            vbuf_v)

            def scatter_body(j):
                ev = plsc.load_gather(vbuf_v, [j * 32 + lane2])
                od = plsc.load_gather(vbuf_v, [j * 32 + lane2 + 1])
                ids = ev | od
                vals = c * VCHUNK + j * LANES + NUM_OOV + lane
                plsc.store_scatter(table_v, [ids], vals)

            _unrolled(VCHUNK // LANES, 5, scatter_body)
            return _

        lax.fori_loop(jnp.int32(0), jnp.int32(v_chunks), build_chunk, None)

        wid = lax.axis_index("s") * 2 + lax.axis_index("c")
        base = wid * jnp.int32(2 * per_w)

        def lookup_chunk(c, _):
            woff = base + c * (2 * CHUNK)
            pltpu.sync_copy(tok_hbm.at[pl.ds(woff, 2 * CHUNK)], inb_v)

            def gather_body(j):
                ev = plsc.load_gather(inb_v, [j * 32 + lane2])
                od = plsc.load_gather(inb_v, [j * 32 + lane2 + 1])
                t = ev | od
                r = plsc.load_gather(table_v, [t])
                plsc.store_scatter(outb_v, [j * 32 + lane2 + off], r)

            _unrolled(CHUNK // LANES, 8, gather_body)
            pltpu.sync_copy(outb_v, out_hbm.at[pl.ds(woff, 2 * CHUNK)])
            return _

        lax.fori_loop(jnp.int32(0), jnp.int32(n_chunks), lookup_chunk, None)

    return k(tokp, vocp, offv)


def kernel(tokens, vocab):
    tokp = lax.bitcast_convert_type(tokens, jnp.int32).reshape(-1)
    vocp = lax.bitcast_convert_type(vocab, jnp.int32).reshape(-1)
    onep = lax.bitcast_convert_type(
        jnp.ones((1,), jnp.int64), jnp.int32).reshape(-1)
    offv = jnp.broadcast_to(jnp.int32(1) - onep[0], (LANES,))
    outp = _sc_lookup(tokp, vocp, offv)
    return lax.bitcast_convert_type(
        outp.reshape(tokens.shape + (2,)), tokens.dtype)


# speedup vs baseline: 10.7593x; 10.7593x over previous
import functools

import jax
import jax.numpy as jnp
from jax import lax
from jax.experimental import pallas as pl
from jax.experimental.pallas import tpu as pltpu
from jax.experimental.pallas import tpu_sc as plsc

TOKEN_UNIVERSE = 120000
TABLE_SIZE = TOKEN_UNIVERSE + 64
NUM_OOV = 1
NUM_WORKERS = 32
LANES = 16
CHUNK_ROWS = 16
VOCAB_PAD = 102400


def _unrolled(n_total, unroll, body):
    assert n_total % unroll == 0

    def outer(o, _):
        for u in range(unroll):
            body(o * jnp.int32(unroll) + jnp.int32(u), u)
        return _

    lax.fori_loop(jnp.int32(0), jnp.int32(n_total // unroll), outer, None)


def _sc_lookup(tok2d, voc2d):
    n_rows = tok2d.shape[0]
    v_rows = voc2d.shape[0]
    rows_w = n_rows // NUM_WORKERS
    n_chunks = rows_w // CHUNK_ROWS
    v_chunks = v_rows // CHUNK_ROWS
    assert rows_w * NUM_WORKERS == n_rows and n_chunks * CHUNK_ROWS == rows_w
    assert v_chunks * CHUNK_ROWS == v_rows

    mesh = plsc.VectorSubcoreMesh(
        core_axis_name="c", subcore_axis_name="s", num_cores=2, num_subcores=16
    )

    @functools.partial(
        pl.kernel,
        out_type=jax.ShapeDtypeStruct((n_rows, 128), jnp.int32),
        mesh=mesh,
        compiler_params=pltpu.CompilerParams(needs_layout_passes=False),
        scratch_types=[
            pltpu.VMEM((TABLE_SIZE,), jnp.int32),
            pltpu.VMEM((CHUNK_ROWS, 128), jnp.int32),
            pltpu.VMEM((CHUNK_ROWS, 128), jnp.int32),
            pltpu.VMEM((CHUNK_ROWS, 128), jnp.int32),
        ],
    )
    def k(tok_hbm, voc_hbm, out_hbm, table_v, vbuf_v, inb_v, outb_v):
        lane = lax.iota(jnp.int32, LANES)

        zeros = jnp.zeros((LANES,), jnp.int32)

        def zero_body(i, _u):
            table_v[pl.ds(i * LANES, LANES)] = zeros

        _unrolled(TABLE_SIZE // LANES, 8, zero_body)

        def build_chunk(c, _):
            pltpu.sync_copy(voc_hbm.at[pl.ds(c * CHUNK_ROWS, CHUNK_ROWS), :],
                            vbuf_v)
            vbase = c * jnp.int32(CHUNK_ROWS * 128) + NUM_OOV + lane

            def scatter_body(j, u):
                row = j >> 3
                ids = vbuf_v[row, pl.ds((u % 8) * LANES, LANES)]
                plsc.store_scatter(table_v, [ids], vbase + j * LANES)

            _unrolled(CHUNK_ROWS * 8, 8, scatter_body)
            return _

        lax.fori_loop(jnp.int32(0), jnp.int32(v_chunks), build_chunk, None)

        wid = lax.axis_index("s") * 2 + lax.axis_index("c")
        base = wid * jnp.int32(rows_w)

        def lookup_chunk(c, _):
            r0 = base + c * CHUNK_ROWS
            pltpu.sync_copy(tok_hbm.at[pl.ds(r0, CHUNK_ROWS), :], inb_v)

            def gather_body(j, u):
                row = j >> 3
                col = (u % 8) * LANES
                t = inb_v[row, pl.ds(col, LANES)]
                outb_v[row, pl.ds(col, LANES)] = plsc.load_gather(table_v, [t])

            _unrolled(CHUNK_ROWS * 8, 8, gather_body)
            pltpu.sync_copy(outb_v, out_hbm.at[pl.ds(r0, CHUNK_ROWS), :])
            return _

        lax.fori_loop(jnp.int32(0), jnp.int32(n_chunks), lookup_chunk, None)

    return k(tok2d, voc2d)


def kernel(tokens, vocab):
    n = tokens.shape[0] * tokens.shape[1]
    tok2d = tokens.astype(jnp.int32).reshape(n // 128, 128)
    voc32 = vocab.astype(jnp.int32)
    pad = jnp.full((VOCAB_PAD - voc32.shape[0],), TOKEN_UNIVERSE, jnp.int32)
    voc2d = jnp.concatenate([voc32, pad]).reshape(VOCAB_PAD // 128, 128)
    out2d = _sc_lookup(tok2d, voc2d)
    return out2d.reshape(tokens.shape).astype(tokens.dtype)


# speedup vs baseline: 11.7318x; 1.0904x over previous
import functools

import jax
import jax.numpy as jnp
from jax import lax
from jax.experimental import pallas as pl
from jax.experimental.pallas import tpu as pltpu
from jax.experimental.pallas import tpu_sc as plsc

TOKEN_UNIVERSE = 120000
TABLE_SIZE = TOKEN_UNIVERSE + 64
NUM_OOV = 1
NUM_WORKERS = 32
LANES = 16
CHUNK_ROWS = 16
VCHUNK_ROWS = 8
VOCAB_PAD = 102400


def _unrolled(n_total, unroll, body):
    assert n_total % unroll == 0

    def outer(o, _):
        for u in range(unroll):
            body(o * jnp.int32(unroll) + jnp.int32(u), u)
        return _

    lax.fori_loop(jnp.int32(0), jnp.int32(n_total // unroll), outer, None)


def _sc_lookup(tok2d, voc2d):
    n_rows = tok2d.shape[0]
    v_rows = voc2d.shape[0]
    rows_w = n_rows // NUM_WORKERS
    n_chunks = rows_w // CHUNK_ROWS
    v_chunks = v_rows // VCHUNK_ROWS
    assert rows_w * NUM_WORKERS == n_rows and n_chunks * CHUNK_ROWS == rows_w
    assert v_chunks * VCHUNK_ROWS == v_rows
    assert n_chunks % 2 == 0 and v_chunks % 2 == 0

    mesh = plsc.VectorSubcoreMesh(
        core_axis_name="c", subcore_axis_name="s", num_cores=2, num_subcores=16
    )

    @functools.partial(
        pl.kernel,
        out_type=jax.ShapeDtypeStruct((n_rows, 128), jnp.int32),
        mesh=mesh,
        compiler_params=pltpu.CompilerParams(needs_layout_passes=False),
        scratch_types=[
            pltpu.VMEM((TABLE_SIZE,), jnp.int32),
            pltpu.VMEM((VCHUNK_ROWS, 128), jnp.int32),
            pltpu.VMEM((VCHUNK_ROWS, 128), jnp.int32),
            pltpu.VMEM((CHUNK_ROWS, 128), jnp.int32),
            pltpu.VMEM((CHUNK_ROWS, 128), jnp.int32),
            pltpu.VMEM((CHUNK_ROWS, 128), jnp.int32),
            pltpu.VMEM((CHUNK_ROWS, 128), jnp.int32),
            pltpu.SemaphoreType.DMA,
            pltpu.SemaphoreType.DMA,
            pltpu.SemaphoreType.DMA,
            pltpu.SemaphoreType.DMA,
            pltpu.SemaphoreType.DMA,
            pltpu.SemaphoreType.DMA,
        ],
    )
    def k(tok_hbm, voc_hbm, out_hbm, table_v, vb0, vb1, ib0, ib1, ob0, ob1,
          sv0, sv1, si0, si1, so0, so1):
        lane = lax.iota(jnp.int32, LANES)
        vbs, ibs, obs = (vb0, vb1), (ib0, ib1), (ob0, ob1)
        svs, sis, sos = (sv0, sv1), (si0, si1), (so0, so1)

        wid = lax.axis_index("s") * 2 + lax.axis_index("c")
        base = wid * jnp.int32(rows_w)

        def vslice(c):
            return voc_hbm.at[pl.ds(c * VCHUNK_ROWS, VCHUNK_ROWS), :]

        def tslice(c):
            return tok_hbm.at[pl.ds(base + c * CHUNK_ROWS, CHUNK_ROWS), :]

        def oslice(c):
            return out_hbm.at[pl.ds(base + c * CHUNK_ROWS, CHUNK_ROWS), :]

        pltpu.async_copy(vslice(jnp.int32(0)), vb0, sv0)
        pltpu.async_copy(tslice(jnp.int32(0)), ib0, si0)

        zeros = jnp.zeros((LANES,), jnp.int32)

        def zero_body(i, _u):
            table_v[pl.ds(i * LANES, LANES)] = zeros

        _unrolled(TABLE_SIZE // LANES, 8, zero_body)

        def build_pair(c2, _):
            for b in (0, 1):
                c = c2 * 2 + jnp.int32(b)
                if b == 0:
                    pltpu.async_copy(vslice(c + 1), vbs[1], svs[1])
                else:
                    @pl.when(c2 < v_chunks // 2 - 1)
                    def _():
                        pltpu.async_copy(vslice(c + 1), vbs[0], svs[0])
                pltpu.make_async_copy(vslice(c), vbs[b], svs[b]).wait()
                vbase = c * jnp.int32(VCHUNK_ROWS * 128) + NUM_OOV + lane

                def scatter_body(j, u, _b=b):
                    row = j >> 3
                    ids = vbs[_b][row, pl.ds((u % 8) * LANES, LANES)]
                    plsc.store_scatter(table_v, [ids], vbase + j * LANES)

                _unrolled(VCHUNK_ROWS * 8, 8, scatter_body)
            return _

        lax.fori_loop(jnp.int32(0), jnp.int32(v_chunks // 2), build_pair, None)

        def lookup_pair(c2, _):
            for b in (0, 1):
                c = c2 * 2 + jnp.int32(b)
                if b == 0:
                    pltpu.async_copy(tslice(c + 1), ibs[1], sis[1])
                else:
                    @pl.when(c2 < n_chunks // 2 - 1)
                    def _():
                        pltpu.async_copy(tslice(c + 1), ibs[0], sis[0])
                pltpu.make_async_copy(tslice(c), ibs[b], sis[b]).wait()

                @pl.when(c2 >= 1)
                def _():
                    pltpu.make_async_copy(obs[b], oslice(c), sos[b]).wait()

                def gather_body(j, u, _b=b):
                    row = j >> 3
                    col = (u % 8) * LANES
                    t = ibs[_b][row, pl.ds(col, LANES)]
                    obs[_b][row, pl.ds(col, LANES)] = plsc.load_gather(
                        table_v, [t])

                _unrolled(CHUNK_ROWS * 8, 8, gather_body)
                pltpu.async_copy(obs[b], oslice(c), sos[b])
            return _

        lax.fori_loop(jnp.int32(0), jnp.int32(n_chunks // 2), lookup_pair, None)

        pltpu.make_async_copy(ob0, oslice(jnp.int32(n_chunks - 2)), so0).wait()
        pltpu.make_async_copy(ob1, oslice(jnp.int32(n_chunks - 1)), so1).wait()

    return k(tok2d, voc2d)


def kernel(tokens, vocab):
    n = tokens.shape[0] * tokens.shape[1]
    tok32 = lax.bitcast_convert_type(tokens.astype(jnp.uint32), jnp.int32)
    tok2d = tok32.reshape(n // 128, 128)
    voc32 = lax.bitcast_convert_type(vocab.astype(jnp.uint32), jnp.int32)
    pad = jnp.full((VOCAB_PAD - voc32.shape[0],), TOKEN_UNIVERSE, jnp.int32)
    voc2d = jnp.concatenate([voc32, pad]).reshape(VOCAB_PAD // 128, 128)
    out2d = _sc_lookup(tok2d, voc2d)
    return out2d.reshape(tokens.shape).astype(tokens.dtype)


# speedup vs baseline: 16.3812x; 1.3963x over previous
import functools

import jax
import jax.numpy as jnp
from jax import lax
from jax.experimental import pallas as pl
from jax.experimental.pallas import tpu as pltpu
from jax.experimental.pallas import tpu_sc as plsc

TOKEN_UNIVERSE = 120000
TABLE_SIZE = TOKEN_UNIVERSE + 64
NUM_OOV = 1
NUM_WORKERS = 32
LANES = 16
COL_W = 512
CHUNK_ROWS = 2
VCHUNK_ROWS = 8
VOCAB_PAD = 102400


def _unrolled(n_total, unroll, body):
    assert n_total % unroll == 0

    def outer(o, _):
        for u in range(unroll):
            body(o * jnp.int32(unroll) + jnp.int32(u), u)
        return _

    lax.fori_loop(jnp.int32(0), jnp.int32(n_total // unroll), outer, None)


def _sc_lookup(tok_t, voc2d):
    t_rows, t_cols = tok_t.shape
    v_rows = voc2d.shape[0]
    n_chunks = t_rows // CHUNK_ROWS
    v_chunks = v_rows // VCHUNK_ROWS
    assert t_cols == COL_W * NUM_WORKERS
    assert n_chunks * CHUNK_ROWS == t_rows
    assert v_chunks * VCHUNK_ROWS == v_rows
    assert n_chunks % 2 == 0 and v_chunks % 2 == 0

    mesh = plsc.VectorSubcoreMesh(
        core_axis_name="c", subcore_axis_name="s", num_cores=2, num_subcores=16
    )

    @functools.partial(
        pl.kernel,
        out_type=jax.ShapeDtypeStruct(tok_t.shape, jnp.int32),
        mesh=mesh,
        compiler_params=pltpu.CompilerParams(
            needs_layout_passes=False, disable_bounds_checks=True),
        scratch_types=[
            pltpu.VMEM((TABLE_SIZE,), jnp.int32),
            pltpu.VMEM((VCHUNK_ROWS, 128), jnp.uint32),
            pltpu.VMEM((VCHUNK_ROWS, 128), jnp.uint32),
            pltpu.VMEM((CHUNK_ROWS, COL_W), jnp.uint32),
            pltpu.VMEM((CHUNK_ROWS, COL_W), jnp.uint32),
            pltpu.VMEM((CHUNK_ROWS, COL_W), jnp.int32),
            pltpu.VMEM((CHUNK_ROWS, COL_W), jnp.int32),
            pltpu.SemaphoreType.DMA,
            pltpu.SemaphoreType.DMA,
            pltpu.SemaphoreType.DMA,
            pltpu.SemaphoreType.DMA,
            pltpu.SemaphoreType.DMA,
            pltpu.SemaphoreType.DMA,
        ],
    )
    def k(tok_hbm, voc_hbm, out_hbm, table_v, vb0, vb1, ib0, ib1, ob0, ob1,
          sv0, sv1, si0, si1, so0, so1):
        lane = lax.iota(jnp.int32, LANES)
        vbs, ibs, obs = (vb0, vb1), (ib0, ib1), (ob0, ob1)
        svs, sis, sos = (sv0, sv1), (si0, si1), (so0, so1)

        wid = lax.axis_index("s") * 2 + lax.axis_index("c")
        col0 = wid * jnp.int32(COL_W)

        def vslice(c):
            return voc_hbm.at[pl.ds(c * VCHUNK_ROWS, VCHUNK_ROWS), :]

        def tslice(c):
            return tok_hbm.at[pl.ds(c * CHUNK_ROWS, CHUNK_ROWS),
                              pl.ds(col0, COL_W)]

        def oslice(c):
            return out_hbm.at[pl.ds(c * CHUNK_ROWS, CHUNK_ROWS),
                              pl.ds(col0, COL_W)]

        pltpu.async_copy(vslice(jnp.int32(0)), vb0, sv0)
        pltpu.async_copy(tslice(jnp.int32(0)), ib0, si0)

        zeros = jnp.zeros((LANES,), jnp.int32)

        def zero_body(i, _u):
            table_v[pl.ds(i * LANES, LANES)] = zeros

        _unrolled(TABLE_SIZE // LANES, 8, zero_body)

        def build_pair(c2, _):
            for b in (0, 1):
                c = c2 * 2 + jnp.int32(b)
                if b == 0:
                    pltpu.async_copy(vslice(c + 1), vbs[1], svs[1])
                else:
                    @pl.when(c2 < v_chunks // 2 - 1)
                    def _():
                        pltpu.async_copy(vslice(c + 1), vbs[0], svs[0])
                pltpu.make_async_copy(vslice(c), vbs[b], svs[b]).wait()
                vbase = c * jnp.int32(VCHUNK_ROWS * 128) + NUM_OOV + lane
                for r in range(VCHUNK_ROWS):
                    for u in range(8):
                        ids = plsc.bitcast(
                            vbs[b][r, pl.ds(u * LANES, LANES)], jnp.int32)
                        plsc.store_scatter(
                            table_v, [ids],
                            vbase + jnp.int32((r * 8 + u) * LANES))
            return _

        lax.fori_loop(jnp.int32(0), jnp.int32(v_chunks // 2), build_pair, None)

        def lookup_pair(c2, _):
            for b in (0, 1):
                c = c2 * 2 + jnp.int32(b)
                if b == 0:
                    pltpu.async_copy(tslice(c + 1), ibs[1], sis[1])
                else:
                    @pl.when(c2 < n_chunks // 2 - 1)
                    def _():
                        pltpu.async_copy(tslice(c + 1), ibs[0], sis[0])
                pltpu.make_async_copy(tslice(c), ibs[b], sis[b]).wait()

                @pl.when(c2 >= 1)
                def _():
                    pltpu.make_async_copy(obs[b], oslice(c), sos[b]).wait()

                for r in range(CHUNK_ROWS):
                    for u in range(COL_W // LANES):
                        t = plsc.bitcast(
                            ibs[b][r, pl.ds(u * LANES, LANES)], jnp.int32)
                        obs[b][r, pl.ds(u * LANES, LANES)] = plsc.load_gather(
                            table_v, [t])
                pltpu.async_copy(obs[b], oslice(c), sos[b])
            return _

        lax.fori_loop(jnp.int32(0), jnp.int32(n_chunks // 2), lookup_pair, None)

        pltpu.make_async_copy(ob0, oslice(jnp.int32(n_chunks - 2)), so0).wait()
        pltpu.make_async_copy(ob1, oslice(jnp.int32(n_chunks - 1)), so1).wait()

    return k(tok_t, voc2d)


def kernel(tokens, vocab):
    tok_t = tokens.T.astype(jnp.uint32)
    voc32 = vocab.astype(jnp.uint32)
    pad = jnp.full((VOCAB_PAD - voc32.shape[0],), TOKEN_UNIVERSE, jnp.uint32)
    voc2d = jnp.concatenate([voc32, pad]).reshape(VOCAB_PAD // 128, 128)
    out_t = _sc_lookup(tok_t, voc2d)
    return out_t.T.astype(tokens.dtype)
